# baseline (speedup 1.0000x reference)
import functools
import math

import jax
import jax.numpy as jnp
from jax import lax
from jax.experimental import pallas as pl
from jax.experimental.pallas import tpu as pltpu

N = 4096
E_PRED = 65536
HIDDEN = 128
NUM_FILTERS = 128
NUM_INT = 6
NUM_G = 50
CUTOFF = 10.0
MAXNB = 32
HALF = HIDDEN // 2
LOG2 = math.log(2.0)


def _ssp(v):
    return jnp.maximum(v, 0.0) + jnp.log(1.0 + jnp.exp(-jnp.abs(v))) - LOG2


_NB_BLK = 128
_NE_BLK = _NB_BLK * MAXNB


def _inter_body(h_ref, ga_ref, wm_ref, hj_ref, w1_ref, b1_ref, w2_ref,
                b2_ref, cf2_ref, cf2b_ref, il_ref, ilb_ref, out_ref):
    t = _ssp(jnp.dot(ga_ref[...], w1_ref[...],
                     preferred_element_type=jnp.float32) + b1_ref[...])
    W = jnp.dot(t, w2_ref[...], preferred_element_type=jnp.float32) + b2_ref[...]
    W = W * wm_ref[...]
    prod = hj_ref[...] * W
    m = jnp.sum(prod.reshape(_NB_BLK, MAXNB, NUM_FILTERS), axis=1)
    m = jnp.dot(m, cf2_ref[...], preferred_element_type=jnp.float32) + cf2b_ref[...]
    m = _ssp(m)
    m = jnp.dot(m, il_ref[...], preferred_element_type=jnp.float32) + ilb_ref[...]
    out_ref[...] = h_ref[...] + m


def _interaction(h, ga_p, wm, hj, w1p, b1, w2, b2, cf2, cf2b, il, ilb):
    grid = N // _NB_BLK
    full = lambda i: (0, 0)
    return pl.pallas_call(
        _inter_body,
        grid=(grid,),
        in_specs=[
            pl.BlockSpec((_NB_BLK, HIDDEN), lambda i: (i, 0)),
            pl.BlockSpec((_NE_BLK, 64), lambda i: (i, 0)),
            pl.BlockSpec((_NE_BLK, 1), lambda i: (i, 0)),
            pl.BlockSpec((_NE_BLK, NUM_FILTERS), lambda i: (i, 0)),
            pl.BlockSpec((64, NUM_FILTERS), full),
            pl.BlockSpec((1, NUM_FILTERS), full),
            pl.BlockSpec((NUM_FILTERS, NUM_FILTERS), full),
            pl.BlockSpec((1, NUM_FILTERS), full),
            pl.BlockSpec((NUM_FILTERS, HIDDEN), full),
            pl.BlockSpec((1, HIDDEN), full),
            pl.BlockSpec((HIDDEN, HIDDEN), full),
            pl.BlockSpec((1, HIDDEN), full),
        ],
        out_specs=pl.BlockSpec((_NB_BLK, HIDDEN), lambda i: (i, 0)),
        out_shape=jax.ShapeDtypeStruct((N, HIDDEN), jnp.float32),
    )(h, ga_p, wm, hj, w1p, b1, w2, b2, cf2, cf2b, il, ilb)


_EE_BLK = 2048
_EDGE_IN_P = 192
_EH_P = 768


def _ehead_body(src_ref, dst_ref, ea_ref, w1_ref, b1_ref, w2_ref, out_ref):
    d = ea_ref[:, 0:1]
    ang = ea_ref[:, 1:2]
    icol = lax.broadcasted_iota(jnp.int32, (_EE_BLK, 64), 1)
    col = icol.astype(jnp.float32)
    step = CUTOFF / (NUM_G - 1)
    coeff = -0.5 / step**2
    g = jnp.exp(coeff * (d - col * step) ** 2)
    g = jnp.where(icol < NUM_G, g, 0.0)
    gext = jnp.where(icol == NUM_G, ang, g)
    ef = jnp.concatenate([src_ref[...], dst_ref[...], gext], axis=1)
    acc = jnp.dot(ef, w1_ref[...], preferred_element_type=jnp.float32) + b1_ref[...]
    v = jax.nn.sigmoid(acc)
    out_ref[...] = jnp.sum(v * w2_ref[...], axis=1, keepdims=True)


def _edge_head(srch, dsth, edge_attr, w1p, b1p, w2p):
    grid = E_PRED // _EE_BLK
    full = lambda i: (0, 0)
    out = pl.pallas_call(
        _ehead_body,
        grid=(grid,),
        in_specs=[
            pl.BlockSpec((_EE_BLK, HALF), lambda i: (i, 0)),
            pl.BlockSpec((_EE_BLK, HALF), lambda i: (i, 0)),
            pl.BlockSpec((_EE_BLK, 2), lambda i: (i, 0)),
            pl.BlockSpec((_EDGE_IN_P, _EH_P), full),
            pl.BlockSpec((1, _EH_P), full),
            pl.BlockSpec((1, _EH_P), full),
        ],
        out_specs=pl.BlockSpec((_EE_BLK, 1), lambda i: (i, 0)),
        out_shape=jax.ShapeDtypeStruct((E_PRED, 1), jnp.float32),
    )(srch, dsth, edge_attr, w1p, b1p, w2p)
    return out[:, 0]


def kernel(x, pos, edge_index, edge_attr, batch, emb, imlp_w1, imlp_b1,
           imlp_w2, imlp_b2, cf1_w, cf2_w, cf2_b, ilin_w, ilin_b, lin1_w,
           lin1_b, emlp1_w, emlp1_b, emlp2_w, emlp2_b):
    h = jnp.take(emb, x, axis=0)

    sq = jnp.sum(pos * pos, axis=1)
    d2 = sq[:, None] + sq[None, :] - 2.0 * (pos @ pos.T)
    invalid = (batch[:, None] != batch[None, :]) | jnp.eye(N, dtype=bool)
    d2 = jnp.where(invalid, 1e12, jnp.maximum(d2, 1e-12))
    neg, nbr = lax.top_k(-d2, MAXNB)
    dist = jnp.sqrt(-neg)
    valid = dist <= CUTOFF
    ew = jnp.where(valid, dist, 0.0)

    offset = jnp.linspace(0.0, CUTOFF, NUM_G)
    coeff = -0.5 / (offset[1] - offset[0]) ** 2
    ga = jnp.exp(coeff * (ew[..., None] - offset) ** 2)
    ga_p = jnp.pad(ga, ((0, 0), (0, 0), (0, 64 - NUM_G))).reshape(N * MAXNB, 64)
    C = 0.5 * (jnp.cos(ew * math.pi / CUTOFF) + 1.0)
    wm = (C * valid.astype(jnp.float32)).reshape(N * MAXNB, 1)

    nbr_flat = nbr.reshape(-1)

    for i in range(NUM_INT):
        w1p = jnp.pad(imlp_w1[i], ((0, 64 - NUM_G), (0, 0)))
        hcf = h @ cf1_w[i]
        hj = jnp.take(hcf, nbr_flat, axis=0)
        h = _interaction(h, ga_p, wm, hj, w1p, imlp_b1[i][None, :],
                         imlp_w2[i], imlp_b2[i][None, :], cf2_w[i],
                         cf2_b[i][None, :], ilin_w[i], ilin_b[i][None, :])

    h2 = h @ lin1_w + lin1_b
    srch = jnp.take(h2, edge_index[0], axis=0)
    dsth = jnp.take(h2, edge_index[1], axis=0)

    w1p = jnp.zeros((_EDGE_IN_P, _EH_P), jnp.float32)
    w1p = w1p.at[:179, :716].set(emlp1_w)
    b1p = jnp.zeros((1, _EH_P), jnp.float32).at[0, :716].set(emlp1_b)
    w2p = jnp.zeros((1, _EH_P), jnp.float32).at[0, :716].set(emlp2_w[:, 0])
    out = _edge_head(srch, dsth, edge_attr, w1p, b1p, w2p)
    return out + emlp2_b[0]


# speedup vs baseline: 2.1331x; 2.1331x over previous
import functools
import math

import jax
import jax.numpy as jnp
from jax import lax
from jax.experimental import pallas as pl
from jax.experimental.pallas import tpu as pltpu

N = 4096
E_PRED = 65536
HIDDEN = 128
NUM_FILTERS = 128
NUM_INT = 6
NUM_G = 50
CUTOFF = 10.0
MAXNB = 32
HALF = HIDDEN // 2
LOG2 = math.log(2.0)


def _ssp(v):
    return jnp.maximum(v, 0.0) + jnp.log(1.0 + jnp.exp(-jnp.abs(v))) - LOG2


_TK_BLK = 128


def _topk_body(posT_ref, posb_ref, brow_ref, bcol_ref, nbr_ref, d2_ref):
    i = pl.program_id(0)
    posT = posT_ref[...]
    posb = posb_ref[...]
    sq_row = jnp.sum(posT * posT, axis=0, keepdims=True)
    sq_col = jnp.sum(posb * posb, axis=1, keepdims=True)
    d2 = sq_col + sq_row - 2.0 * jnp.dot(
        posb, posT, preferred_element_type=jnp.float32)
    colv = lax.broadcasted_iota(jnp.int32, (_TK_BLK, N), 1)
    rowv = lax.broadcasted_iota(jnp.int32, (_TK_BLK, N), 0) + i * _TK_BLK
    invalid = (bcol_ref[...] != brow_ref[...]) | (colv == rowv)
    vals = jnp.where(invalid, 1e12, jnp.maximum(d2, 1e-12))
    for k in range(MAXNB):
        rowmin = jnp.min(vals, axis=1, keepdims=True)
        idx = jnp.min(jnp.where(vals == rowmin, colv, N),
                      axis=1, keepdims=True)
        nbr_ref[:, k:k + 1] = idx
        d2_ref[:, k:k + 1] = rowmin
        vals = jnp.where(colv == idx, 1e12, vals)


def _radius_topk(pos, batch):
    posT = jnp.zeros((8, N), jnp.float32).at[:3].set(pos.T)
    posb = jnp.zeros((N, 8), jnp.float32).at[:, :3].set(pos)
    brow = batch.astype(jnp.int32).reshape(1, N)
    bcol = batch.astype(jnp.int32).reshape(N, 1)
    grid = N // _TK_BLK
    nbr, d2 = pl.pallas_call(
        _topk_body,
        grid=(grid,),
        in_specs=[
            pl.BlockSpec((8, N), lambda i: (0, 0)),
            pl.BlockSpec((_TK_BLK, 8), lambda i: (i, 0)),
            pl.BlockSpec((1, N), lambda i: (0, 0)),
            pl.BlockSpec((_TK_BLK, 1), lambda i: (i, 0)),
        ],
        out_specs=[
            pl.BlockSpec((_TK_BLK, MAXNB), lambda i: (i, 0)),
            pl.BlockSpec((_TK_BLK, MAXNB), lambda i: (i, 0)),
        ],
        out_shape=[
            jax.ShapeDtypeStruct((N, MAXNB), jnp.int32),
            jax.ShapeDtypeStruct((N, MAXNB), jnp.float32),
        ],
    )(posT, posb, brow, bcol)
    return nbr, d2


_NB_BLK = 128
_NE_BLK = _NB_BLK * MAXNB


def _inter_body(h_ref, ga_ref, wm_ref, hj_ref, w1_ref, b1_ref, w2_ref,
                b2_ref, cf2_ref, cf2b_ref, il_ref, ilb_ref, out_ref):
    t = _ssp(jnp.dot(ga_ref[...], w1_ref[...],
                     preferred_element_type=jnp.float32) + b1_ref[...])
    W = jnp.dot(t, w2_ref[...], preferred_element_type=jnp.float32) + b2_ref[...]
    W = W * wm_ref[...]
    prod = hj_ref[...] * W
    m = jnp.sum(prod.reshape(_NB_BLK, MAXNB, NUM_FILTERS), axis=1)
    m = jnp.dot(m, cf2_ref[...], preferred_element_type=jnp.float32) + cf2b_ref[...]
    m = _ssp(m)
    m = jnp.dot(m, il_ref[...], preferred_element_type=jnp.float32) + ilb_ref[...]
    out_ref[...] = h_ref[...] + m


def _interaction(h, ga_p, wm, hj, w1p, b1, w2, b2, cf2, cf2b, il, ilb):
    grid = N // _NB_BLK
    full = lambda i: (0, 0)
    return pl.pallas_call(
        _inter_body,
        grid=(grid,),
        in_specs=[
            pl.BlockSpec((_NB_BLK, HIDDEN), lambda i: (i, 0)),
            pl.BlockSpec((_NE_BLK, 64), lambda i: (i, 0)),
            pl.BlockSpec((_NE_BLK, 1), lambda i: (i, 0)),
            pl.BlockSpec((_NE_BLK, NUM_FILTERS), lambda i: (i, 0)),
            pl.BlockSpec((64, NUM_FILTERS), full),
            pl.BlockSpec((1, NUM_FILTERS), full),
            pl.BlockSpec((NUM_FILTERS, NUM_FILTERS), full),
            pl.BlockSpec((1, NUM_FILTERS), full),
            pl.BlockSpec((NUM_FILTERS, HIDDEN), full),
            pl.BlockSpec((1, HIDDEN), full),
            pl.BlockSpec((HIDDEN, HIDDEN), full),
            pl.BlockSpec((1, HIDDEN), full),
        ],
        out_specs=pl.BlockSpec((_NB_BLK, HIDDEN), lambda i: (i, 0)),
        out_shape=jax.ShapeDtypeStruct((N, HIDDEN), jnp.float32),
    )(h, ga_p, wm, hj, w1p, b1, w2, b2, cf2, cf2b, il, ilb)


_EE_BLK = 2048
_EDGE_IN_P = 192
_EH_P = 768


def _ehead_body(src_ref, dst_ref, ea_ref, w1_ref, b1_ref, w2_ref, out_ref):
    d = ea_ref[:, 0:1]
    ang = ea_ref[:, 1:2]
    icol = lax.broadcasted_iota(jnp.int32, (_EE_BLK, 64), 1)
    col = icol.astype(jnp.float32)
    step = CUTOFF / (NUM_G - 1)
    coeff = -0.5 / step**2
    g = jnp.exp(coeff * (d - col * step) ** 2)
    g = jnp.where(icol < NUM_G, g, 0.0)
    gext = jnp.where(icol == NUM_G, ang, g)
    ef = jnp.concatenate([src_ref[...], dst_ref[...], gext], axis=1)
    acc = jnp.dot(ef, w1_ref[...], preferred_element_type=jnp.float32) + b1_ref[...]
    v = jax.nn.sigmoid(acc)
    out_ref[...] = jnp.sum(v * w2_ref[...], axis=1, keepdims=True)


def _edge_head(srch, dsth, edge_attr, w1p, b1p, w2p):
    grid = E_PRED // _EE_BLK
    full = lambda i: (0, 0)
    out = pl.pallas_call(
        _ehead_body,
        grid=(grid,),
        in_specs=[
            pl.BlockSpec((_EE_BLK, HALF), lambda i: (i, 0)),
            pl.BlockSpec((_EE_BLK, HALF), lambda i: (i, 0)),
            pl.BlockSpec((_EE_BLK, 2), lambda i: (i, 0)),
            pl.BlockSpec((_EDGE_IN_P, _EH_P), full),
            pl.BlockSpec((1, _EH_P), full),
            pl.BlockSpec((1, _EH_P), full),
        ],
        out_specs=pl.BlockSpec((_EE_BLK, 1), lambda i: (i, 0)),
        out_shape=jax.ShapeDtypeStruct((E_PRED, 1), jnp.float32),
    )(srch, dsth, edge_attr, w1p, b1p, w2p)
    return out[:, 0]


def kernel(x, pos, edge_index, edge_attr, batch, emb, imlp_w1, imlp_b1,
           imlp_w2, imlp_b2, cf1_w, cf2_w, cf2_b, ilin_w, ilin_b, lin1_w,
           lin1_b, emlp1_w, emlp1_b, emlp2_w, emlp2_b):
    h = jnp.take(emb, x, axis=0)

    nbr, d2min = _radius_topk(pos, batch)
    dist = jnp.sqrt(d2min)
    valid = dist <= CUTOFF
    ew = jnp.where(valid, dist, 0.0)

    offset = jnp.linspace(0.0, CUTOFF, NUM_G)
    coeff = -0.5 / (offset[1] - offset[0]) ** 2
    ga = jnp.exp(coeff * (ew[..., None] - offset) ** 2)
    ga_p = jnp.pad(ga, ((0, 0), (0, 0), (0, 64 - NUM_G))).reshape(N * MAXNB, 64)
    C = 0.5 * (jnp.cos(ew * math.pi / CUTOFF) + 1.0)
    wm = (C * valid.astype(jnp.float32)).reshape(N * MAXNB, 1)

    nbr_flat = nbr.reshape(-1)

    for i in range(NUM_INT):
        w1p = jnp.pad(imlp_w1[i], ((0, 64 - NUM_G), (0, 0)))
        hcf = h @ cf1_w[i]
        hj = jnp.take(hcf, nbr_flat, axis=0)
        h = _interaction(h, ga_p, wm, hj, w1p, imlp_b1[i][None, :],
                         imlp_w2[i], imlp_b2[i][None, :], cf2_w[i],
                         cf2_b[i][None, :], ilin_w[i], ilin_b[i][None, :])

    h2 = h @ lin1_w + lin1_b
    srch = jnp.take(h2, edge_index[0], axis=0)
    dsth = jnp.take(h2, edge_index[1], axis=0)

    w1p = jnp.zeros((_EDGE_IN_P, _EH_P), jnp.float32)
    w1p = w1p.at[:179, :716].set(emlp1_w)
    b1p = jnp.zeros((1, _EH_P), jnp.float32).at[0, :716].set(emlp1_b)
    w2p = jnp.zeros((1, _EH_P), jnp.float32).at[0, :716].set(emlp2_w[:, 0])
    out = _edge_head(srch, dsth, edge_attr, w1p, b1p, w2p)
    return out + emlp2_b[0]


# speedup vs baseline: 5.4112x; 2.5368x over previous
import functools
import math

import jax
import jax.numpy as jnp
from jax import lax
from jax.experimental import pallas as pl
from jax.experimental.pallas import tpu as pltpu
from jax.experimental.pallas import tpu_sc as plsc

N = 4096
E_PRED = 65536
HIDDEN = 128
NUM_FILTERS = 128
NUM_INT = 6
NUM_G = 50
CUTOFF = 10.0
MAXNB = 32
HALF = HIDDEN // 2
LOG2 = math.log(2.0)


def _ssp(v):
    return jnp.maximum(v, 0.0) + jnp.log(1.0 + jnp.exp(-jnp.abs(v))) - LOG2


_TK_BLK = 128


def _topk_body(posT_ref, posb_ref, brow_ref, bcol_ref, nbr_ref, d2_ref):
    i = pl.program_id(0)
    posT = posT_ref[...]
    posb = posb_ref[...]
    sq_row = jnp.sum(posT * posT, axis=0, keepdims=True)
    sq_col = jnp.sum(posb * posb, axis=1, keepdims=True)
    d2 = sq_col + sq_row - 2.0 * jnp.dot(
        posb, posT, preferred_element_type=jnp.float32)
    colv = lax.broadcasted_iota(jnp.int32, (_TK_BLK, N), 1)
    rowv = lax.broadcasted_iota(jnp.int32, (_TK_BLK, N), 0) + i * _TK_BLK
    invalid = (bcol_ref[...] != brow_ref[...]) | (colv == rowv)
    vals = jnp.where(invalid, 1e12, jnp.maximum(d2, 1e-12))
    for k in range(MAXNB):
        rowmin = jnp.min(vals, axis=1, keepdims=True)
        idx = jnp.min(jnp.where(vals == rowmin, colv, N),
                      axis=1, keepdims=True)
        nbr_ref[:, k:k + 1] = idx
        d2_ref[:, k:k + 1] = rowmin
        vals = jnp.where(colv == idx, 1e12, vals)


def _radius_topk(pos, batch):
    posT = jnp.zeros((8, N), jnp.float32).at[:3].set(pos.T)
    posb = jnp.zeros((N, 8), jnp.float32).at[:, :3].set(pos)
    brow = batch.astype(jnp.int32).reshape(1, N)
    bcol = batch.astype(jnp.int32).reshape(N, 1)
    grid = N // _TK_BLK
    nbr, d2 = pl.pallas_call(
        _topk_body,
        grid=(grid,),
        in_specs=[
            pl.BlockSpec((8, N), lambda i: (0, 0)),
            pl.BlockSpec((_TK_BLK, 8), lambda i: (i, 0)),
            pl.BlockSpec((1, N), lambda i: (0, 0)),
            pl.BlockSpec((_TK_BLK, 1), lambda i: (i, 0)),
        ],
        out_specs=[
            pl.BlockSpec((_TK_BLK, MAXNB), lambda i: (i, 0)),
            pl.BlockSpec((_TK_BLK, MAXNB), lambda i: (i, 0)),
        ],
        out_shape=[
            jax.ShapeDtypeStruct((N, MAXNB), jnp.int32),
            jax.ShapeDtypeStruct((N, MAXNB), jnp.float32),
        ],
    )(posT, posb, brow, bcol)
    return nbr, d2


_SC_NC = 2
_SC_NS = 16
_SC_NW = _SC_NC * _SC_NS
_SC_CH = 128


def _sc_gather(table, idx):
    B = idx.shape[0]
    D = table.shape[1]
    b_per_w = B // _SC_NW
    n_ch = b_per_w // _SC_CH
    mesh = plsc.VectorSubcoreMesh(core_axis_name="c", subcore_axis_name="s")

    @functools.partial(
        pl.kernel,
        mesh=mesh,
        out_type=jax.ShapeDtypeStruct((B, D), jnp.float32),
        scratch_types=[
            pltpu.VMEM((b_per_w,), jnp.int32),
            pltpu.VMEM((2, _SC_CH, D), jnp.float32),
            pltpu.SemaphoreType.DMA,
            pltpu.SemaphoreType.DMA,
        ],
    )
    def gk(table_hbm, idx_hbm, out_hbm, idx_v, rows_v, sem0, sem1):
        wid = lax.axis_index("s") * _SC_NC + lax.axis_index("c")
        base = wid * b_per_w
        pltpu.sync_copy(idx_hbm.at[pl.ds(base, b_per_w)], idx_v)
        sems = [sem0, sem1]
        for j in range(2):
            pltpu.make_async_copy(
                table_hbm.at[idx_v.at[pl.ds(j * _SC_CH, _SC_CH)]],
                rows_v.at[j], sems[j]).start()

        def body(g, _):
            for j in range(2):
                ci = g * 2 + j
                pltpu.make_async_copy(
                    table_hbm.at[idx_v.at[pl.ds(ci * _SC_CH, _SC_CH)]],
                    rows_v.at[j], sems[j]).wait()
                pltpu.sync_copy(
                    rows_v.at[j],
                    out_hbm.at[pl.ds(base + ci * _SC_CH, _SC_CH)])

                @pl.when(ci + 2 < n_ch)
                def _():
                    pltpu.make_async_copy(
                        table_hbm.at[idx_v.at[pl.ds((ci + 2) * _SC_CH, _SC_CH)]],
                        rows_v.at[j], sems[j]).start()

            return ()

        lax.fori_loop(0, n_ch // 2, body, ())

    return gk(table, idx)


_NB_BLK = 128
_NE_BLK = _NB_BLK * MAXNB


def _inter_body(h_ref, ga_ref, wm_ref, hj_ref, w1_ref, b1_ref, w2_ref,
                b2_ref, cf2_ref, cf2b_ref, il_ref, ilb_ref, out_ref):
    t = _ssp(jnp.dot(ga_ref[...], w1_ref[...],
                     preferred_element_type=jnp.float32) + b1_ref[...])
    W = jnp.dot(t, w2_ref[...], preferred_element_type=jnp.float32) + b2_ref[...]
    W = W * wm_ref[...]
    prod = hj_ref[...] * W
    m = jnp.sum(prod.reshape(_NB_BLK, MAXNB, NUM_FILTERS), axis=1)
    m = jnp.dot(m, cf2_ref[...], preferred_element_type=jnp.float32) + cf2b_ref[...]
    m = _ssp(m)
    m = jnp.dot(m, il_ref[...], preferred_element_type=jnp.float32) + ilb_ref[...]
    out_ref[...] = h_ref[...] + m


def _interaction(h, ga_p, wm, hj, w1p, b1, w2, b2, cf2, cf2b, il, ilb):
    grid = N // _NB_BLK
    full = lambda i: (0, 0)
    return pl.pallas_call(
        _inter_body,
        grid=(grid,),
        in_specs=[
            pl.BlockSpec((_NB_BLK, HIDDEN), lambda i: (i, 0)),
            pl.BlockSpec((_NE_BLK, 64), lambda i: (i, 0)),
            pl.BlockSpec((_NE_BLK, 1), lambda i: (i, 0)),
            pl.BlockSpec((_NE_BLK, NUM_FILTERS), lambda i: (i, 0)),
            pl.BlockSpec((64, NUM_FILTERS), full),
            pl.BlockSpec((1, NUM_FILTERS), full),
            pl.BlockSpec((NUM_FILTERS, NUM_FILTERS), full),
            pl.BlockSpec((1, NUM_FILTERS), full),
            pl.BlockSpec((NUM_FILTERS, HIDDEN), full),
            pl.BlockSpec((1, HIDDEN), full),
            pl.BlockSpec((HIDDEN, HIDDEN), full),
            pl.BlockSpec((1, HIDDEN), full),
        ],
        out_specs=pl.BlockSpec((_NB_BLK, HIDDEN), lambda i: (i, 0)),
        out_shape=jax.ShapeDtypeStruct((N, HIDDEN), jnp.float32),
    )(h, ga_p, wm, hj, w1p, b1, w2, b2, cf2, cf2b, il, ilb)


_EE_BLK = 2048
_EDGE_IN_P = 192
_EH_P = 768


def _ehead_body(src_ref, dst_ref, ea_ref, w1_ref, b1_ref, w2_ref, out_ref):
    d = ea_ref[:, 0:1]
    ang = ea_ref[:, 1:2]
    icol = lax.broadcasted_iota(jnp.int32, (_EE_BLK, 64), 1)
    col = icol.astype(jnp.float32)
    step = CUTOFF / (NUM_G - 1)
    coeff = -0.5 / step**2
    g = jnp.exp(coeff * (d - col * step) ** 2)
    g = jnp.where(icol < NUM_G, g, 0.0)
    gext = jnp.where(icol == NUM_G, ang, g)
    ef = jnp.concatenate([src_ref[...], dst_ref[...], gext], axis=1)
    acc = jnp.dot(ef, w1_ref[...], preferred_element_type=jnp.float32) + b1_ref[...]
    v = jax.nn.sigmoid(acc)
    out_ref[...] = jnp.sum(v * w2_ref[...], axis=1, keepdims=True)


def _edge_head(srch, dsth, edge_attr, w1p, b1p, w2p):
    grid = E_PRED // _EE_BLK
    full = lambda i: (0, 0)
    out = pl.pallas_call(
        _ehead_body,
        grid=(grid,),
        in_specs=[
            pl.BlockSpec((_EE_BLK, HALF), lambda i: (i, 0)),
            pl.BlockSpec((_EE_BLK, HALF), lambda i: (i, 0)),
            pl.BlockSpec((_EE_BLK, 2), lambda i: (i, 0)),
            pl.BlockSpec((_EDGE_IN_P, _EH_P), full),
            pl.BlockSpec((1, _EH_P), full),
            pl.BlockSpec((1, _EH_P), full),
        ],
        out_specs=pl.BlockSpec((_EE_BLK, 1), lambda i: (i, 0)),
        out_shape=jax.ShapeDtypeStruct((E_PRED, 1), jnp.float32),
    )(srch, dsth, edge_attr, w1p, b1p, w2p)
    return out[:, 0]


def kernel(x, pos, edge_index, edge_attr, batch, emb, imlp_w1, imlp_b1,
           imlp_w2, imlp_b2, cf1_w, cf2_w, cf2_b, ilin_w, ilin_b, lin1_w,
           lin1_b, emlp1_w, emlp1_b, emlp2_w, emlp2_b):
    h = jnp.take(emb, x, axis=0)

    nbr, d2min = _radius_topk(pos, batch)
    dist = jnp.sqrt(d2min)
    valid = dist <= CUTOFF
    ew = jnp.where(valid, dist, 0.0)

    offset = jnp.linspace(0.0, CUTOFF, NUM_G)
    coeff = -0.5 / (offset[1] - offset[0]) ** 2
    ga = jnp.exp(coeff * (ew[..., None] - offset) ** 2)
    ga_p = jnp.pad(ga, ((0, 0), (0, 0), (0, 64 - NUM_G))).reshape(N * MAXNB, 64)
    C = 0.5 * (jnp.cos(ew * math.pi / CUTOFF) + 1.0)
    wm = (C * valid.astype(jnp.float32)).reshape(N * MAXNB, 1)

    nbr_flat = nbr.reshape(-1)

    for i in range(NUM_INT):
        w1p = jnp.pad(imlp_w1[i], ((0, 64 - NUM_G), (0, 0)))
        hcf = h @ cf1_w[i]
        hj = _sc_gather(hcf, nbr_flat)
        h = _interaction(h, ga_p, wm, hj, w1p, imlp_b1[i][None, :],
                         imlp_w2[i], imlp_b2[i][None, :], cf2_w[i],
                         cf2_b[i][None, :], ilin_w[i], ilin_b[i][None, :])

    h2 = h @ lin1_w + lin1_b
    h2p = jnp.pad(h2, ((0, 0), (0, HIDDEN - HALF)))
    sd = _sc_gather(h2p, edge_index.reshape(-1))
    srch = sd[:E_PRED, :HALF]
    dsth = sd[E_PRED:, :HALF]

    w1p = jnp.zeros((_EDGE_IN_P, _EH_P), jnp.float32)
    w1p = w1p.at[:179, :716].set(emlp1_w)
    b1p = jnp.zeros((1, _EH_P), jnp.float32).at[0, :716].set(emlp1_b)
    w2p = jnp.zeros((1, _EH_P), jnp.float32).at[0, :716].set(emlp2_w[:, 0])
    out = _edge_head(srch, dsth, edge_attr, w1p, b1p, w2p)
    return out + emlp2_b[0]
